# baseline (speedup 1.0000x reference)
import functools

import jax
import jax.numpy as jnp
from jax import lax
from jax.experimental import pallas as pl
from jax.experimental.pallas import tpu as pltpu
from jax.experimental.pallas import tpu_sc as plsc

KNN = 16
BQ = 512

_NC, _NS = 2, 16
_NW = _NC * _NS

_DEPTH = 5
_LANES = 128


def _stage1_body(f_ref, fq_ref, w1d_ref, w1b_ref, b1_ref,
                 idx_ref, u_ref, v_ref):
    i = pl.program_id(0)
    f = f_ref[...]
    fq = fq_ref[...]
    n = f.shape[0]
    bq = fq.shape[0]

    sq_all = jnp.sum(f * f, axis=1)[:, None]
    sq_q = jnp.sum(fq * fq, axis=1)[None, :]
    prod = lax.dot_general(f, fq, (((1,), (1,)), ((), ())),
                           preferred_element_type=jnp.float32)
    d2 = sq_all + sq_q - 2.0 * prod
    dist = jnp.sqrt(jnp.maximum(d2, 0.0))

    col = lax.broadcasted_iota(jnp.int32, (n, bq), 0)
    qcol = i * bq + lax.broadcasted_iota(jnp.int32, (n, bq), 1)
    inf = jnp.float32(jnp.inf)
    dist = jnp.where(col == qcol, inf, dist)

    n_ch = n // _LANES
    s_val = [jnp.full((_LANES, bq), inf, jnp.float32) for _ in range(_DEPTH)]
    s_idx = [jnp.zeros((_LANES, bq), jnp.int32) for _ in range(_DEPTH)]
    for v in range(n_ch):
        carry = dist[v * _LANES:(v + 1) * _LANES, :]
        carryi = jnp.full((_LANES, bq), v, jnp.int32)
        for k in range(_DEPTH):
            lower = carry < s_val[k]
            new_v = jnp.where(lower, carry, s_val[k])
            carry = jnp.where(lower, s_val[k], carry)
            new_i = jnp.where(lower, carryi, s_idx[k])
            carryi = jnp.where(lower, s_idx[k], carryi)
            s_val[k], s_idx[k] = new_v, new_i

    lane_io = lax.broadcasted_iota(jnp.int32, (_LANES, bq), 0)
    cnt = jnp.zeros((_LANES, bq), jnp.int32)
    idx_rows = []
    for _ in range(KNN):
        m = jnp.min(s_val[0], axis=0, keepdims=True)
        colc = s_idx[0] * _LANES + lane_io
        cand = jnp.where(s_val[0] <= m, colc, n)
        amin = jnp.min(cand, axis=0, keepdims=True)
        idx_rows.append(amin)
        hit = colc == amin
        for k in range(_DEPTH - 1):
            s_val[k] = jnp.where(hit, s_val[k + 1], s_val[k])
            s_idx[k] = jnp.where(hit, s_idx[k + 1], s_idx[k])
        s_val[_DEPTH - 1] = jnp.where(hit, inf, s_val[_DEPTH - 1])
        cnt = cnt + hit.astype(jnp.int32)
    idx_ref[...] = jnp.concatenate(idx_rows, axis=0)

    @pl.when(jnp.any(cnt >= _DEPTH))
    def _fallback():
        d = dist
        rows = []
        for _ in range(KNN):
            mm = jnp.min(d, axis=0, keepdims=True)
            cc = jnp.where(d <= mm, col, n)
            aa = jnp.min(cc, axis=0, keepdims=True)
            rows.append(aa)
            d = jnp.where(col == aa, inf, d)
        idx_ref[...] = jnp.concatenate(rows, axis=0)

    u_ref[...] = (jnp.dot(fq, w1d_ref[...], preferred_element_type=jnp.float32)
                  + b1_ref[...])
    vdot = jnp.dot(fq, w1b_ref[...], preferred_element_type=jnp.float32)
    v_ref[...] = jnp.concatenate(
        [vdot, jnp.zeros_like(vdot)], axis=1)


def _gelu_exact(x):
    return 0.5 * x * (1.0 + lax.erf(x * jnp.float32(0.7071067811865476)))


def _stage3_body(u_ref, vg_ref, w2t_ref, b2_ref, o_ref):
    u = u_ref[...]
    bq, out = u.shape
    vg = vg_ref[...][:, :, :out]
    h1 = _gelu_exact(vg + u[None, :, :])
    h2 = (jnp.dot(h1.reshape(KNN * bq, out), w2t_ref[...],
                  preferred_element_type=jnp.float32) + b2_ref[...])
    h2 = _gelu_exact(h2)
    o_ref[...] = jnp.max(h2.reshape(KNN, bq, out), axis=0)


def _sc_gather(table, idx):
    v_rows, d = table.shape
    b = idx.shape[0]
    b_per_w = b // _NW
    ch = 512
    n_ch = b_per_w // ch
    mesh = plsc.VectorSubcoreMesh(core_axis_name="c", subcore_axis_name="s")

    @functools.partial(
        pl.kernel, mesh=mesh,
        out_type=jax.ShapeDtypeStruct((b, d), jnp.float32),
        scratch_types=[
            pltpu.VMEM((ch,), jnp.int32),
            pltpu.VMEM((ch, d), jnp.float32),
            pltpu.SemaphoreType.DMA,
        ],
    )
    def k(table_hbm, idx_hbm, out_hbm, idx_v, rows_v, sem):
        wid = lax.axis_index("s") * _NC + lax.axis_index("c")
        base = wid * b_per_w

        @pl.loop(0, n_ch)
        def _(c):
            off = base + c * ch
            pltpu.sync_copy(idx_hbm.at[pl.ds(off, ch)], idx_v)
            pltpu.async_copy(table_hbm.at[idx_v], rows_v, sem).wait()
            pltpu.sync_copy(rows_v, out_hbm.at[pl.ds(off, ch)])

    return k(table, idx)


def kernel(features, W1, b1, W2, b2):
    batch, n, c = features.shape
    out_dim = W1.shape[0]
    f = features.reshape(n, c)
    w1a = W1[:, :c]
    w1b = W1[:, c:]
    w1d_t = (w1a - w1b).T
    w1b_t = w1b.T
    b1r = b1.reshape(1, out_dim)
    w2t = W2.T
    b2r = b2.reshape(1, out_dim)

    n_blocks = n // BQ
    idx, u, v = pl.pallas_call(
        _stage1_body,
        grid=(n_blocks,),
        in_specs=[
            pl.BlockSpec((n, c), lambda i: (0, 0)),
            pl.BlockSpec((BQ, c), lambda i: (i, 0)),
            pl.BlockSpec((c, out_dim), lambda i: (0, 0)),
            pl.BlockSpec((c, out_dim), lambda i: (0, 0)),
            pl.BlockSpec((1, out_dim), lambda i: (0, 0)),
        ],
        out_specs=[
            pl.BlockSpec((KNN, BQ), lambda i: (0, i)),
            pl.BlockSpec((BQ, out_dim), lambda i: (i, 0)),
            pl.BlockSpec((BQ, 2 * out_dim), lambda i: (i, 0)),
        ],
        out_shape=[
            jax.ShapeDtypeStruct((KNN, n), jnp.int32),
            jax.ShapeDtypeStruct((n, out_dim), jnp.float32),
            jax.ShapeDtypeStruct((n, 2 * out_dim), jnp.float32),
        ],
        compiler_params=pltpu.CompilerParams(
            dimension_semantics=("parallel",)),
    )(f, f, w1d_t, w1b_t, b1r)

    vg = _sc_gather(v, idx.reshape(KNN * n))
    vg3 = vg.reshape(KNN, n, 2 * out_dim)

    out = pl.pallas_call(
        _stage3_body,
        grid=(n_blocks,),
        in_specs=[
            pl.BlockSpec((BQ, out_dim), lambda i: (i, 0)),
            pl.BlockSpec((KNN, BQ, 2 * out_dim), lambda i: (0, i, 0)),
            pl.BlockSpec((out_dim, out_dim), lambda i: (0, 0)),
            pl.BlockSpec((1, out_dim), lambda i: (0, 0)),
        ],
        out_specs=pl.BlockSpec((BQ, out_dim), lambda i: (i, 0)),
        out_shape=jax.ShapeDtypeStruct((n, out_dim), jnp.float32),
        compiler_params=pltpu.CompilerParams(
            dimension_semantics=("parallel",)),
    )(u, vg3, w2t, b2r)

    return out.reshape(batch, n, out_dim)


# speedup vs baseline: 1.1685x; 1.1685x over previous
import functools

import jax
import jax.numpy as jnp
from jax import lax
from jax.experimental import pallas as pl
from jax.experimental.pallas import tpu as pltpu
from jax.experimental.pallas import tpu_sc as plsc

KNN = 16
BQ = 128

_NC, _NS = 2, 16
_NW = _NC * _NS

_DEPTH = 5
_LANES = 128


def _stage1_body(f_ref, fq_ref, w1d_ref, w1b_ref, b1_ref,
                 idx_ref, u_ref, v_ref):
    i = pl.program_id(0)
    f = f_ref[...]
    fq = fq_ref[...]
    n = f.shape[0]
    bq = fq.shape[0]

    sq_all = jnp.sum(f * f, axis=1)[:, None]
    sq_q = jnp.sum(fq * fq, axis=1)[None, :]
    prod = lax.dot_general(f, fq, (((1,), (1,)), ((), ())),
                           preferred_element_type=jnp.float32)
    d2 = sq_all + sq_q - 2.0 * prod
    dist = jnp.sqrt(jnp.maximum(d2, 0.0))

    col = lax.broadcasted_iota(jnp.int32, (n, bq), 0)
    qcol = i * bq + lax.broadcasted_iota(jnp.int32, (n, bq), 1)
    inf = jnp.float32(jnp.inf)
    dist = jnp.where(col == qcol, inf, dist)

    n_ch = n // _LANES
    s_val = [jnp.full((_LANES, bq), inf, jnp.float32) for _ in range(_DEPTH)]
    s_idx = [jnp.zeros((_LANES, bq), jnp.int32) for _ in range(_DEPTH)]
    for v in range(n_ch):
        carry = dist[v * _LANES:(v + 1) * _LANES, :]
        carryi = jnp.full((_LANES, bq), v, jnp.int32)
        for k in range(_DEPTH):
            lower = carry < s_val[k]
            new_v = jnp.where(lower, carry, s_val[k])
            carry = jnp.where(lower, s_val[k], carry)
            new_i = jnp.where(lower, carryi, s_idx[k])
            carryi = jnp.where(lower, s_idx[k], carryi)
            s_val[k], s_idx[k] = new_v, new_i

    lane_io = lax.broadcasted_iota(jnp.int32, (_LANES, bq), 0)
    cnt = jnp.zeros((_LANES, bq), jnp.int32)
    idx_rows = []
    for _ in range(KNN):
        m = jnp.min(s_val[0], axis=0, keepdims=True)
        colc = s_idx[0] * _LANES + lane_io
        cand = jnp.where(s_val[0] <= m, colc, n)
        amin = jnp.min(cand, axis=0, keepdims=True)
        idx_rows.append(amin)
        hit = colc == amin
        for k in range(_DEPTH - 1):
            s_val[k] = jnp.where(hit, s_val[k + 1], s_val[k])
            s_idx[k] = jnp.where(hit, s_idx[k + 1], s_idx[k])
        s_val[_DEPTH - 1] = jnp.where(hit, inf, s_val[_DEPTH - 1])
        cnt = cnt + hit.astype(jnp.int32)
    idx_ref[...] = jnp.concatenate(idx_rows, axis=0)

    @pl.when(jnp.any(cnt >= _DEPTH))
    def _fallback():
        d = dist
        rows = []
        for _ in range(KNN):
            mm = jnp.min(d, axis=0, keepdims=True)
            cc = jnp.where(d <= mm, col, n)
            aa = jnp.min(cc, axis=0, keepdims=True)
            rows.append(aa)
            d = jnp.where(col == aa, inf, d)
        idx_ref[...] = jnp.concatenate(rows, axis=0)

    u_ref[...] = (jnp.dot(fq, w1d_ref[...], preferred_element_type=jnp.float32)
                  + b1_ref[...])
    vdot = jnp.dot(fq, w1b_ref[...], preferred_element_type=jnp.float32)
    v_ref[...] = jnp.concatenate(
        [vdot, jnp.zeros_like(vdot)], axis=1)


def _gelu_exact(x):
    return 0.5 * x * (1.0 + lax.erf(x * jnp.float32(0.7071067811865476)))


def _stage3_body(u_ref, vg_ref, w2t_ref, b2_ref, o_ref):
    u = u_ref[...]
    bq, out = u.shape
    vg = vg_ref[...][:, :, :out]
    h1 = _gelu_exact(vg + u[None, :, :])
    h2 = (jnp.dot(h1.reshape(KNN * bq, out), w2t_ref[...],
                  preferred_element_type=jnp.float32) + b2_ref[...])
    h2 = _gelu_exact(h2)
    o_ref[...] = jnp.max(h2.reshape(KNN, bq, out), axis=0)


def _sc_gather(table, idx):
    v_rows, d = table.shape
    b = idx.shape[0]
    b_per_w = b // _NW
    ch = 512
    n_ch = b_per_w // ch
    mesh = plsc.VectorSubcoreMesh(core_axis_name="c", subcore_axis_name="s")

    @functools.partial(
        pl.kernel, mesh=mesh,
        out_type=jax.ShapeDtypeStruct((b, d), jnp.float32),
        scratch_types=[
            pltpu.VMEM((ch,), jnp.int32),
            pltpu.VMEM((ch, d), jnp.float32),
            pltpu.SemaphoreType.DMA,
        ],
    )
    def k(table_hbm, idx_hbm, out_hbm, idx_v, rows_v, sem):
        wid = lax.axis_index("s") * _NC + lax.axis_index("c")
        base = wid * b_per_w

        @pl.loop(0, n_ch)
        def _(c):
            off = base + c * ch
            pltpu.sync_copy(idx_hbm.at[pl.ds(off, ch)], idx_v)
            pltpu.async_copy(table_hbm.at[idx_v], rows_v, sem).wait()
            pltpu.sync_copy(rows_v, out_hbm.at[pl.ds(off, ch)])

    return k(table, idx)


def kernel(features, W1, b1, W2, b2):
    batch, n, c = features.shape
    out_dim = W1.shape[0]
    f = features.reshape(n, c)
    w1a = W1[:, :c]
    w1b = W1[:, c:]
    w1d_t = (w1a - w1b).T
    w1b_t = w1b.T
    b1r = b1.reshape(1, out_dim)
    w2t = W2.T
    b2r = b2.reshape(1, out_dim)

    n_blocks = n // BQ
    idx, u, v = pl.pallas_call(
        _stage1_body,
        grid=(n_blocks,),
        in_specs=[
            pl.BlockSpec((n, c), lambda i: (0, 0)),
            pl.BlockSpec((BQ, c), lambda i: (i, 0)),
            pl.BlockSpec((c, out_dim), lambda i: (0, 0)),
            pl.BlockSpec((c, out_dim), lambda i: (0, 0)),
            pl.BlockSpec((1, out_dim), lambda i: (0, 0)),
        ],
        out_specs=[
            pl.BlockSpec((KNN, BQ), lambda i: (0, i)),
            pl.BlockSpec((BQ, out_dim), lambda i: (i, 0)),
            pl.BlockSpec((BQ, 2 * out_dim), lambda i: (i, 0)),
        ],
        out_shape=[
            jax.ShapeDtypeStruct((KNN, n), jnp.int32),
            jax.ShapeDtypeStruct((n, out_dim), jnp.float32),
            jax.ShapeDtypeStruct((n, 2 * out_dim), jnp.float32),
        ],
        compiler_params=pltpu.CompilerParams(
            dimension_semantics=("parallel",)),
    )(f, f, w1d_t, w1b_t, b1r)

    vg = _sc_gather(v, idx.reshape(KNN * n))
    vg3 = vg.reshape(KNN, n, 2 * out_dim)

    out = pl.pallas_call(
        _stage3_body,
        grid=(n_blocks,),
        in_specs=[
            pl.BlockSpec((BQ, out_dim), lambda i: (i, 0)),
            pl.BlockSpec((KNN, BQ, 2 * out_dim), lambda i: (0, i, 0)),
            pl.BlockSpec((out_dim, out_dim), lambda i: (0, 0)),
            pl.BlockSpec((1, out_dim), lambda i: (0, 0)),
        ],
        out_specs=pl.BlockSpec((BQ, out_dim), lambda i: (i, 0)),
        out_shape=jax.ShapeDtypeStruct((n, out_dim), jnp.float32),
        compiler_params=pltpu.CompilerParams(
            dimension_semantics=("parallel",)),
    )(u, vg3, w2t, b2r)

    return out.reshape(batch, n, out_dim)


# speedup vs baseline: 1.2481x; 1.0682x over previous
import functools

import jax
import jax.numpy as jnp
from jax import lax
from jax.experimental import pallas as pl
from jax.experimental.pallas import tpu as pltpu
from jax.experimental.pallas import tpu_sc as plsc

KNN = 16
BQ = 256

_NC, _NS = 2, 16
_NW = _NC * _NS

_DEPTH = 5
_LANES = 128


def _stage1_body(f_ref, fq_ref, w1d_ref, w1b_ref, b1_ref,
                 idx_ref, u_ref, v_ref):
    i = pl.program_id(0)
    f = f_ref[...]
    fq = fq_ref[...]
    n = f.shape[0]
    bq = fq.shape[0]

    sq_all = jnp.sum(f * f, axis=1)[:, None]
    sq_q = jnp.sum(fq * fq, axis=1)[None, :]
    prod = lax.dot_general(f, fq, (((1,), (1,)), ((), ())),
                           preferred_element_type=jnp.float32)
    d2 = sq_all + sq_q - 2.0 * prod
    dist = jnp.sqrt(jnp.maximum(d2, 0.0))

    col = lax.broadcasted_iota(jnp.int32, (n, bq), 0)
    qcol = i * bq + lax.broadcasted_iota(jnp.int32, (n, bq), 1)
    inf = jnp.float32(jnp.inf)
    dist = jnp.where(col == qcol, inf, dist)

    n_ch = n // _LANES
    s_val = [jnp.full((_LANES, bq), inf, jnp.float32) for _ in range(_DEPTH)]
    s_idx = [jnp.zeros((_LANES, bq), jnp.int32) for _ in range(_DEPTH)]
    for v in range(n_ch):
        carry = dist[v * _LANES:(v + 1) * _LANES, :]
        carryi = jnp.full((_LANES, bq), v, jnp.int32)
        for k in range(_DEPTH):
            lower = carry < s_val[k]
            new_v = jnp.where(lower, carry, s_val[k])
            carry = jnp.where(lower, s_val[k], carry)
            new_i = jnp.where(lower, carryi, s_idx[k])
            carryi = jnp.where(lower, s_idx[k], carryi)
            s_val[k], s_idx[k] = new_v, new_i

    lane_io = lax.broadcasted_iota(jnp.int32, (_LANES, bq), 0)
    cnt = jnp.zeros((_LANES, bq), jnp.int32)
    idx_rows = []
    for _ in range(KNN):
        m = jnp.min(s_val[0], axis=0, keepdims=True)
        colc = s_idx[0] * _LANES + lane_io
        cand = jnp.where(s_val[0] <= m, colc, n)
        amin = jnp.min(cand, axis=0, keepdims=True)
        idx_rows.append(amin)
        hit = colc == amin
        for k in range(_DEPTH - 1):
            s_val[k] = jnp.where(hit, s_val[k + 1], s_val[k])
            s_idx[k] = jnp.where(hit, s_idx[k + 1], s_idx[k])
        s_val[_DEPTH - 1] = jnp.where(hit, inf, s_val[_DEPTH - 1])
        cnt = cnt + hit.astype(jnp.int32)
    idx_ref[...] = jnp.concatenate(idx_rows, axis=0)

    @pl.when(jnp.any(cnt >= _DEPTH))
    def _fallback():
        d = dist
        rows = []
        for _ in range(KNN):
            mm = jnp.min(d, axis=0, keepdims=True)
            cc = jnp.where(d <= mm, col, n)
            aa = jnp.min(cc, axis=0, keepdims=True)
            rows.append(aa)
            d = jnp.where(col == aa, inf, d)
        idx_ref[...] = jnp.concatenate(rows, axis=0)

    u_ref[...] = (jnp.dot(fq, w1d_ref[...], preferred_element_type=jnp.float32)
                  + b1_ref[...])
    vdot = jnp.dot(fq, w1b_ref[...], preferred_element_type=jnp.float32)
    v_ref[...] = jnp.concatenate(
        [vdot, jnp.zeros_like(vdot)], axis=1)


def _gelu_exact(x):
    return 0.5 * x * (1.0 + lax.erf(x * jnp.float32(0.7071067811865476)))


def _stage3_body(u_ref, vg_ref, w2t_ref, b2_ref, o_ref):
    u = u_ref[...]
    bq, out = u.shape
    vg = vg_ref[...][:, :, :out]
    h1 = _gelu_exact(vg + u[None, :, :])
    h2 = (jnp.dot(h1.reshape(KNN * bq, out), w2t_ref[...],
                  preferred_element_type=jnp.float32) + b2_ref[...])
    h2 = _gelu_exact(h2)
    o_ref[...] = jnp.max(h2.reshape(KNN, bq, out), axis=0)


def _sc_gather(table, idx):
    v_rows, d = table.shape
    b = idx.shape[0]
    b_per_w = b // _NW
    ch = 512
    n_ch = b_per_w // ch
    mesh = plsc.VectorSubcoreMesh(core_axis_name="c", subcore_axis_name="s")

    @functools.partial(
        pl.kernel, mesh=mesh,
        out_type=jax.ShapeDtypeStruct((b, d), jnp.float32),
        scratch_types=[
            pltpu.VMEM((ch,), jnp.int32),
            pltpu.VMEM((ch, d), jnp.float32),
            pltpu.SemaphoreType.DMA,
        ],
    )
    def k(table_hbm, idx_hbm, out_hbm, idx_v, rows_v, sem):
        wid = lax.axis_index("s") * _NC + lax.axis_index("c")
        base = wid * b_per_w

        @pl.loop(0, n_ch)
        def _(c):
            off = base + c * ch
            pltpu.sync_copy(idx_hbm.at[pl.ds(off, ch)], idx_v)
            pltpu.async_copy(table_hbm.at[idx_v], rows_v, sem).wait()
            pltpu.sync_copy(rows_v, out_hbm.at[pl.ds(off, ch)])

    return k(table, idx)


def kernel(features, W1, b1, W2, b2):
    batch, n, c = features.shape
    out_dim = W1.shape[0]
    f = features.reshape(n, c)
    w1a = W1[:, :c]
    w1b = W1[:, c:]
    w1d_t = (w1a - w1b).T
    w1b_t = w1b.T
    b1r = b1.reshape(1, out_dim)
    w2t = W2.T
    b2r = b2.reshape(1, out_dim)

    n_blocks = n // BQ
    idx, u, v = pl.pallas_call(
        _stage1_body,
        grid=(n_blocks,),
        in_specs=[
            pl.BlockSpec((n, c), lambda i: (0, 0)),
            pl.BlockSpec((BQ, c), lambda i: (i, 0)),
            pl.BlockSpec((c, out_dim), lambda i: (0, 0)),
            pl.BlockSpec((c, out_dim), lambda i: (0, 0)),
            pl.BlockSpec((1, out_dim), lambda i: (0, 0)),
        ],
        out_specs=[
            pl.BlockSpec((KNN, BQ), lambda i: (0, i)),
            pl.BlockSpec((BQ, out_dim), lambda i: (i, 0)),
            pl.BlockSpec((BQ, 2 * out_dim), lambda i: (i, 0)),
        ],
        out_shape=[
            jax.ShapeDtypeStruct((KNN, n), jnp.int32),
            jax.ShapeDtypeStruct((n, out_dim), jnp.float32),
            jax.ShapeDtypeStruct((n, 2 * out_dim), jnp.float32),
        ],
        compiler_params=pltpu.CompilerParams(
            dimension_semantics=("parallel",)),
    )(f, f, w1d_t, w1b_t, b1r)

    vg = _sc_gather(v, idx.reshape(KNN * n))
    vg3 = vg.reshape(KNN, n, 2 * out_dim)

    out = pl.pallas_call(
        _stage3_body,
        grid=(n_blocks,),
        in_specs=[
            pl.BlockSpec((BQ, out_dim), lambda i: (i, 0)),
            pl.BlockSpec((KNN, BQ, 2 * out_dim), lambda i: (0, i, 0)),
            pl.BlockSpec((out_dim, out_dim), lambda i: (0, 0)),
            pl.BlockSpec((1, out_dim), lambda i: (0, 0)),
        ],
        out_specs=pl.BlockSpec((BQ, out_dim), lambda i: (i, 0)),
        out_shape=jax.ShapeDtypeStruct((n, out_dim), jnp.float32),
        compiler_params=pltpu.CompilerParams(
            dimension_semantics=("parallel",)),
    )(u, vg3, w2t, b2r)

    return out.reshape(batch, n, out_dim)
